# baseline (speedup 1.0000x reference)
import functools

import jax
import jax.numpy as jnp
from jax import lax
import jax.experimental.pallas as pl
from jax.experimental.pallas import tpu as pltpu
from jax.experimental.pallas import tpu_sc as plsc

BATCH = 128
IN_F = 4096
R = 64
OUT_F = 4096
E = 64

EB = 8
JB = 512

_WORKERS = 32
_CH = 16


def _copy_rows(src, dst, row0, nrows, buf, sem_l, sem_s):
    nch = nrows // _CH

    def load(k, slot):
        return pltpu.make_async_copy(
            src.at[pl.ds(row0 + k * _CH, _CH)], buf.at[slot],
            sem_l.at[slot])

    def store(k, slot):
        return pltpu.make_async_copy(
            buf.at[slot], dst.at[pl.ds(row0 + k * _CH, _CH)],
            sem_s.at[slot])

    load(0, 0).start()
    for k in range(nch):
        slot = k % 2
        if k + 1 < nch:
            if k >= 1:
                store(k - 1, (k + 1) % 2).wait()
            load(k + 1, (k + 1) % 2).start()
        load(k, slot).wait()
        store(k, slot).start()
    if nch >= 2:
        store(nch - 2, (nch - 2) % 2).wait()
    store(nch - 1, (nch - 1) % 2).wait()


def _sc_ingest_one(t_ref, o_ref, buf, sem_l, sem_s):
    c = lax.axis_index("c")
    s = lax.axis_index("s")
    idx = c * lax.axis_size("s") + s
    rows = t_ref.shape[0] // _WORKERS
    _copy_rows(t_ref.bitcast(jnp.int16), o_ref, idx * rows, rows,
               buf, sem_l, sem_s)


def _decode_words(w):
    t = ((w + 0x00040004) >> 3) & 0x0FFF0FFF
    return (t + 0x38003800) | (w & jnp.int32(-2147450880))


def _decode16(h16):
    w = pltpu.bitcast(h16, jnp.int32)
    return pltpu.bitcast(_decode_words(w), jnp.bfloat16)


def _stage1(wids_ref, x_ref, a_ref, u_ref):
    g = pl.program_id(0)
    x = x_ref[...]
    wids = wids_ref[...]
    at = jnp.concatenate([_decode16(a_ref[e]) for e in range(EB)], axis=0)
    u = jax.lax.dot_general(x, at, (((1,), (1,)), ((), ())),
                            preferred_element_type=jnp.float32)
    col_adapter = g * EB + jax.lax.broadcasted_iota(
        jnp.int32, (BATCH, EB * R), 1) // R
    u_ref[...] = jnp.where(wids == col_adapter, u, 0.0).astype(jnp.bfloat16)


def _stage2m(x_ref, m_ref, o_ref):
    m = _decode16(m_ref[...])
    o_ref[...] = jnp.dot(x_ref[...], m, preferred_element_type=jnp.float32)


def _stage2b(u_ref, b_ref, o_ref):
    b = _decode16(b_ref[...])
    o_ref[...] = 2.0 * jnp.dot(u_ref[...], b,
                               preferred_element_type=jnp.float32)


def kernel(x, wids, lora_A, lora_B, M):
    x2 = x.reshape(BATCH, IN_F).astype(jnp.bfloat16)
    wids2 = wids.reshape(BATCH, 1)
    a_t = jnp.transpose(lora_A, (0, 2, 1)).reshape(E * R, IN_F)
    b_flat = lora_B.reshape(E * R, OUT_F)

    def sc_ingest(nrows, ncols):
        return functools.partial(
            pl.kernel,
            out_type=jax.ShapeDtypeStruct((nrows, ncols), jnp.int16),
            mesh=plsc.VectorSubcoreMesh(core_axis_name="c",
                                        subcore_axis_name="s"),
            scratch_types=[
                pltpu.VMEM((2, _CH, ncols), jnp.int16),
                pltpu.SemaphoreType.DMA((2,)),
                pltpu.SemaphoreType.DMA((2,)),
            ],
        )(_sc_ingest_one)

    a_i16 = sc_ingest(E * R, IN_F)(a_t).reshape(E, R, IN_F)
    m_i16 = sc_ingest(IN_F, OUT_F)(M)
    b_i16 = sc_ingest(E * R, OUT_F)(b_flat)

    u = pl.pallas_call(
        _stage1,
        grid=(E // EB,),
        in_specs=[
            pl.BlockSpec((BATCH, 1), lambda g: (0, 0)),
            pl.BlockSpec((BATCH, IN_F), lambda g: (0, 0)),
            pl.BlockSpec((EB, R, IN_F), lambda g: (g, 0, 0)),
        ],
        out_specs=pl.BlockSpec((BATCH, EB * R), lambda g: (0, g)),
        out_shape=jax.ShapeDtypeStruct((BATCH, E * R), jnp.bfloat16),
    )(wids2, x2, a_i16)

    y_m = pl.pallas_call(
        _stage2m,
        grid=(OUT_F // JB,),
        in_specs=[
            pl.BlockSpec((BATCH, IN_F), lambda j: (0, 0)),
            pl.BlockSpec((IN_F, JB), lambda j: (0, j)),
        ],
        out_specs=pl.BlockSpec((BATCH, JB), lambda j: (0, j)),
        out_shape=jax.ShapeDtypeStruct((BATCH, OUT_F), jnp.float32),
    )(x2, m_i16)

    y_b = pl.pallas_call(
        _stage2b,
        grid=(OUT_F // JB,),
        in_specs=[
            pl.BlockSpec((BATCH, E * R), lambda j: (0, 0)),
            pl.BlockSpec((E * R, JB), lambda j: (0, j)),
        ],
        out_specs=pl.BlockSpec((BATCH, JB), lambda j: (0, j)),
        out_shape=jax.ShapeDtypeStruct((BATCH, OUT_F), jnp.float32),
    )(u, b_i16)

    y = y_m + y_b
    return y.reshape(BATCH, 1, OUT_F).astype(jnp.float16)


# speedup vs baseline: 1.0160x; 1.0160x over previous
import functools

import jax
import jax.numpy as jnp
from jax import lax
import jax.experimental.pallas as pl
from jax.experimental.pallas import tpu as pltpu
from jax.experimental.pallas import tpu_sc as plsc

BATCH = 128
IN_F = 4096
R = 64
OUT_F = 4096
E = 64

EB = 8
JB = 512

_WORKERS = 32
_CH = 16


def _copy_rows(src, dst, row0, nrows, buf, sem_l, sem_s):
    nch = nrows // _CH

    def load(k, slot):
        return pltpu.make_async_copy(
            src.at[pl.ds(row0 + k * _CH, _CH)], buf.at[slot],
            sem_l.at[slot])

    def store(k, slot):
        return pltpu.make_async_copy(
            buf.at[slot], dst.at[pl.ds(row0 + k * _CH, _CH)],
            sem_s.at[slot])

    load(0, 0).start()
    load(1, 1).start()
    for k in range(nch):
        slot = k % 3
        if k + 2 < nch:
            if k >= 1:
                store(k - 1, (k + 2) % 3).wait()
            load(k + 2, (k + 2) % 3).start()
        load(k, slot).wait()
        store(k, slot).start()
    for t in range(max(0, nch - 3), nch):
        store(t, t % 3).wait()


def _sc_ingest_one(t_ref, o_ref, buf, sem_l, sem_s):
    c = lax.axis_index("c")
    s = lax.axis_index("s")
    idx = c * lax.axis_size("s") + s
    rows = t_ref.shape[0] // _WORKERS
    _copy_rows(t_ref.bitcast(jnp.int16), o_ref, idx * rows, rows,
               buf, sem_l, sem_s)


def _decode_words(w):
    t = ((w + 0x00040004) >> 3) & 0x0FFF0FFF
    return (t + 0x38003800) | (w & jnp.int32(-2147450880))


def _decode16(h16):
    w = pltpu.bitcast(h16, jnp.int32)
    return pltpu.bitcast(_decode_words(w), jnp.bfloat16)


def _stage1(wids_ref, x_ref, a_ref, u_ref):
    g = pl.program_id(0)
    x = x_ref[...]
    wids = wids_ref[...]
    at = jnp.concatenate([_decode16(a_ref[e]) for e in range(EB)], axis=0)
    u = jax.lax.dot_general(x, at, (((1,), (1,)), ((), ())),
                            preferred_element_type=jnp.float32)
    col_adapter = g * EB + jax.lax.broadcasted_iota(
        jnp.int32, (BATCH, EB * R), 1) // R
    u_ref[...] = jnp.where(wids == col_adapter, u, 0.0).astype(jnp.bfloat16)


def _stage2m(x_ref, m_ref, o_ref):
    m = _decode16(m_ref[...])
    o_ref[...] = jnp.dot(x_ref[...], m, preferred_element_type=jnp.float32)


def _stage2b(u_ref, b_ref, o_ref):
    b = _decode16(b_ref[...])
    o_ref[...] = 2.0 * jnp.dot(u_ref[...], b,
                               preferred_element_type=jnp.float32)


def kernel(x, wids, lora_A, lora_B, M):
    x2 = x.reshape(BATCH, IN_F).astype(jnp.bfloat16)
    wids2 = wids.reshape(BATCH, 1)
    a_t = jnp.transpose(lora_A, (0, 2, 1)).reshape(E * R, IN_F)
    b_flat = lora_B.reshape(E * R, OUT_F)

    def sc_ingest(nrows, ncols):
        return functools.partial(
            pl.kernel,
            out_type=jax.ShapeDtypeStruct((nrows, ncols), jnp.int16),
            mesh=plsc.VectorSubcoreMesh(core_axis_name="c",
                                        subcore_axis_name="s"),
            scratch_types=[
                pltpu.VMEM((3, _CH, ncols), jnp.int16),
                pltpu.SemaphoreType.DMA((3,)),
                pltpu.SemaphoreType.DMA((3,)),
            ],
        )(_sc_ingest_one)

    a_i16 = sc_ingest(E * R, IN_F)(a_t).reshape(E, R, IN_F)
    m_i16 = sc_ingest(IN_F, OUT_F)(M)
    b_i16 = sc_ingest(E * R, OUT_F)(b_flat)

    u = pl.pallas_call(
        _stage1,
        grid=(E // EB,),
        in_specs=[
            pl.BlockSpec((BATCH, 1), lambda g: (0, 0)),
            pl.BlockSpec((BATCH, IN_F), lambda g: (0, 0)),
            pl.BlockSpec((EB, R, IN_F), lambda g: (g, 0, 0)),
        ],
        out_specs=pl.BlockSpec((BATCH, EB * R), lambda g: (0, g)),
        out_shape=jax.ShapeDtypeStruct((BATCH, E * R), jnp.bfloat16),
    )(wids2, x2, a_i16)

    y_m = pl.pallas_call(
        _stage2m,
        grid=(OUT_F // JB,),
        in_specs=[
            pl.BlockSpec((BATCH, IN_F), lambda j: (0, 0)),
            pl.BlockSpec((IN_F, JB), lambda j: (0, j)),
        ],
        out_specs=pl.BlockSpec((BATCH, JB), lambda j: (0, j)),
        out_shape=jax.ShapeDtypeStruct((BATCH, OUT_F), jnp.float32),
    )(x2, m_i16)

    y_b = pl.pallas_call(
        _stage2b,
        grid=(OUT_F // JB,),
        in_specs=[
            pl.BlockSpec((BATCH, E * R), lambda j: (0, 0)),
            pl.BlockSpec((E * R, JB), lambda j: (0, j)),
        ],
        out_specs=pl.BlockSpec((BATCH, JB), lambda j: (0, j)),
        out_shape=jax.ShapeDtypeStruct((BATCH, OUT_F), jnp.float32),
    )(u, b_i16)

    y = y_m + y_b
    return y.reshape(BATCH, 1, OUT_F).astype(jnp.float16)


# speedup vs baseline: 1.0238x; 1.0077x over previous
import functools

import jax
import jax.numpy as jnp
from jax import lax
import jax.experimental.pallas as pl
from jax.experimental.pallas import tpu as pltpu
from jax.experimental.pallas import tpu_sc as plsc

BATCH = 128
IN_F = 4096
R = 64
OUT_F = 4096
E = 64

EB = 8
JB = 512

_WORKERS = 32
_CH = 16


def _copy_rows(src, dst, row0, nrows, buf, sem_l, sem_s):
    nch = nrows // _CH

    def load(k, slot):
        return pltpu.make_async_copy(
            src.at[pl.ds(row0 + k * _CH, _CH)], buf.at[slot],
            sem_l.at[slot])

    def store(k, slot):
        return pltpu.make_async_copy(
            buf.at[slot], dst.at[pl.ds(row0 + k * _CH, _CH)],
            sem_s.at[slot])

    load(0, 0).start()
    load(1, 1).start()
    for k in range(nch):
        slot = k % 3
        if k + 2 < nch:
            if k >= 1:
                store(k - 1, (k + 2) % 3).wait()
            load(k + 2, (k + 2) % 3).start()
        load(k, slot).wait()
        store(k, slot).start()
    for t in range(max(0, nch - 3), nch):
        store(t, t % 3).wait()


def _sc_ingest_one(t_ref, o_ref, buf, sem_l, sem_s):
    c = lax.axis_index("c")
    s = lax.axis_index("s")
    idx = c * lax.axis_size("s") + s
    rows = t_ref.shape[0] // _WORKERS
    _copy_rows(t_ref.bitcast(jnp.int16), o_ref, idx * rows, rows,
               buf, sem_l, sem_s)


def _sc_ingest_two(m_ref, b_ref, om_ref, ob_ref, buf, sem_l, sem_s):
    c = lax.axis_index("c")
    s = lax.axis_index("s")
    idx = c * lax.axis_size("s") + s
    rm = m_ref.shape[0] // _WORKERS
    _copy_rows(m_ref.bitcast(jnp.int16), om_ref, idx * rm, rm,
               buf, sem_l, sem_s)
    rb = b_ref.shape[0] // _WORKERS
    _copy_rows(b_ref.bitcast(jnp.int16), ob_ref, idx * rb, rb,
               buf, sem_l, sem_s)


def _decode_words(w):
    t = ((w + 0x00040004) >> 3) & 0x0FFF0FFF
    return (t + 0x38003800) | (w & jnp.int32(-2147450880))


def _decode16(h16):
    w = pltpu.bitcast(h16, jnp.int32)
    return pltpu.bitcast(_decode_words(w), jnp.bfloat16)


def _stage1(wids_ref, x_ref, a_ref, u_ref):
    g = pl.program_id(0)
    x = x_ref[...]
    wids = wids_ref[...]
    at = jnp.concatenate([_decode16(a_ref[e]) for e in range(EB)], axis=0)
    u = jax.lax.dot_general(x, at, (((1,), (1,)), ((), ())),
                            preferred_element_type=jnp.float32)
    col_adapter = g * EB + jax.lax.broadcasted_iota(
        jnp.int32, (BATCH, EB * R), 1) // R
    u_ref[...] = jnp.where(wids == col_adapter, u, 0.0).astype(jnp.bfloat16)


def _stage2(x_ref, u_ref, m_ref, b_ref, o_ref):
    m = _decode16(m_ref[...])
    b = _decode16(b_ref[...])
    acc = jnp.dot(x_ref[...], m, preferred_element_type=jnp.float32)
    acc += 2.0 * jnp.dot(u_ref[...], b, preferred_element_type=jnp.float32)
    o_ref[...] = acc


def kernel(x, wids, lora_A, lora_B, M):
    x2 = x.reshape(BATCH, IN_F).astype(jnp.bfloat16)
    wids2 = wids.reshape(BATCH, 1)
    a_t = jnp.transpose(lora_A, (0, 2, 1)).reshape(E * R, IN_F)
    b_flat = lora_B.reshape(E * R, OUT_F)

    sc_one = functools.partial(
        pl.kernel,
        out_type=jax.ShapeDtypeStruct((E * R, IN_F), jnp.int16),
        mesh=plsc.VectorSubcoreMesh(core_axis_name="c", subcore_axis_name="s"),
        scratch_types=[
            pltpu.VMEM((3, _CH, IN_F), jnp.int16),
            pltpu.SemaphoreType.DMA((3,)),
            pltpu.SemaphoreType.DMA((3,)),
        ],
    )(_sc_ingest_one)
    sc_two = functools.partial(
        pl.kernel,
        out_type=(
            jax.ShapeDtypeStruct((IN_F, OUT_F), jnp.int16),
            jax.ShapeDtypeStruct((E * R, OUT_F), jnp.int16),
        ),
        mesh=plsc.VectorSubcoreMesh(core_axis_name="c", subcore_axis_name="s"),
        scratch_types=[
            pltpu.VMEM((3, _CH, OUT_F), jnp.int16),
            pltpu.SemaphoreType.DMA((3,)),
            pltpu.SemaphoreType.DMA((3,)),
        ],
    )(_sc_ingest_two)

    a_i16 = sc_one(a_t).reshape(E, R, IN_F)
    m_i16, b_i16 = sc_two(M, b_flat)

    u = pl.pallas_call(
        _stage1,
        grid=(E // EB,),
        in_specs=[
            pl.BlockSpec((BATCH, 1), lambda g: (0, 0)),
            pl.BlockSpec((BATCH, IN_F), lambda g: (0, 0)),
            pl.BlockSpec((EB, R, IN_F), lambda g: (g, 0, 0)),
        ],
        out_specs=pl.BlockSpec((BATCH, EB * R), lambda g: (0, g)),
        out_shape=jax.ShapeDtypeStruct((BATCH, E * R), jnp.bfloat16),
    )(wids2, x2, a_i16)

    y = pl.pallas_call(
        _stage2,
        grid=(OUT_F // JB,),
        in_specs=[
            pl.BlockSpec((BATCH, IN_F), lambda j: (0, 0)),
            pl.BlockSpec((BATCH, E * R), lambda j: (0, 0)),
            pl.BlockSpec((IN_F, JB), lambda j: (0, j)),
            pl.BlockSpec((E * R, JB), lambda j: (0, j)),
        ],
        out_specs=pl.BlockSpec((BATCH, JB), lambda j: (0, j)),
        out_shape=jax.ShapeDtypeStruct((BATCH, OUT_F), jnp.float32),
    )(x2, u, m_i16, b_i16)

    return y.reshape(BATCH, 1, OUT_F).astype(jnp.float16)
